# baseline (speedup 1.0000x reference)
import functools
import math

import jax
import jax.numpy as jnp
import numpy as np
from jax import lax
from jax.experimental import pallas as pl
from jax.experimental.pallas import tpu as pltpu
from jax.experimental.pallas import tpu_sc as plsc

N_ATOMS = 100000
N_MOL = 4096
SQ3 = 1.0 / math.sqrt(3.0)

NW = 32
NPAD = 102400
CHUNK = NPAD // NW
BN = 3200
NBLK = NPAD // BN
IDX_CH = 128
N_IDX_CH = CHUNK // IDX_CH

_SMAT = np.kron(np.eye(16, dtype=np.float32), np.ones((5, 1), np.float32))
_SMAT_T = np.ascontiguousarray(_SMAT.T)

_M8 = np.zeros((8, 1), np.float32)
_M8[1:6, 0] = 1.0
_AMAT = np.zeros((8, 9), np.float32)
_AMAT[0, [0, 4, 8]] = 1.0
_AMAT[1, [1, 3]] = 1.0
_AMAT[2, [5, 7]] = 1.0
_AMAT[3, [0, 4, 8]] = [-SQ3, -SQ3, 2.0 * SQ3]
_AMAT[4, [2, 6]] = 1.0
_AMAT[5, [0, 4]] = [1.0, -1.0]
_BVEC = np.zeros((1, 9), np.float32)
_BVEC[0, [0, 4, 8]] = SQ3


def _dense_body(xs_ref, x0_ref, x2_ref, sw1_ref, sw2_ref, pw0_ref,
                w2pad_ref, s_ref, st_ref, qw0_ref, q2big_ref,
                sb1_ref, sb2_ref, pb0_ref, qb0_ref, out_ref):
    pid = pl.program_id(0)

    h = xs_ref[...] @ sw1_ref[...] + sb1_ref[...]
    h = h * jax.nn.sigmoid(h)
    so = h @ sw2_ref[...] + sb2_ref[...]

    h0 = (x0_ref[...] @ pw0_ref[...]) * (1.0 / math.sqrt(128.0)) + pb0_ref[...]
    h0 = h0 * jax.nn.sigmoid(jnp.abs(h0))

    lane = lax.broadcasted_iota(jnp.int32, (BN, 256), 1)
    x2 = jnp.where(lane < 224, x2_ref[...], 0.0)
    h2 = x2 @ w2pad_ref[...]
    nsq = (h2 * h2) @ s_ref[...]
    g = jax.nn.sigmoid(jnp.sqrt(nsq + 1e-12))
    h2 = h2 * (g @ st_ref[...])

    o0 = (h0 @ qw0_ref[...]) * (1.0 / math.sqrt(64.0)) + qb0_ref[...]
    o2 = h2 @ q2big_ref[...]
    a0 = o0 * so[:, 0:1]
    a2 = o2 * so[:, 1:2]
    out = jnp.concatenate(
        [a0, a2, jnp.zeros((BN, 2), jnp.float32)], axis=-1)
    row = pid * BN + lax.broadcasted_iota(jnp.int32, (BN, 8), 0)
    out_ref[...] = jnp.where(row < N_ATOMS, out, 0.0)


def _whole(shape):
    return pl.BlockSpec(shape, lambda i: tuple(0 for _ in shape))


_dense_call = pl.pallas_call(
    _dense_body,
    grid=(NBLK,),
    in_specs=[
        pl.BlockSpec((BN, 128), lambda i: (i, 0)),
        pl.BlockSpec((BN, 128), lambda i: (i, 0)),
        pl.BlockSpec((BN, 256), lambda i: (i, 1)),
        _whole((128, 64)),
        _whole((64, 2)),
        _whole((128, 64)),
        _whole((256, 80)),
        _whole((80, 16)),
        _whole((16, 80)),
        _whole((64, 1)),
        _whole((80, 5)),
        _whole((1, 64)),
        _whole((1, 2)),
        _whole((1, 64)),
        _whole((1, 1)),
    ],
    out_specs=pl.BlockSpec((BN, 8), lambda i: (i, 0)),
    out_shape=jax.ShapeDtypeStruct((NPAD, 8), jnp.float32),
)


@functools.partial(
    pl.kernel,
    out_type=jax.ShapeDtypeStruct((2, N_MOL, 8), jnp.float32),
    mesh=plsc.VectorSubcoreMesh(core_axis_name="c", subcore_axis_name="s"),
    compiler_params=pltpu.CompilerParams(use_tc_tiling_on_sc=False),
    scratch_types=[
        pltpu.VMEM((N_IDX_CH, IDX_CH), jnp.int32),
        pltpu.VMEM((CHUNK, 8), jnp.float32),
        pltpu.VMEM_SHARED((N_MOL, 8), jnp.float32),
        pltpu.SemaphoreType.DMA,
        pltpu.SemaphoreType.DMA,
    ],
)
def _segsum(vals_hbm, idx_hbm, zeros_hbm, out_hbm, idx_v, vals_v, acc_sh,
            ld_sem, sc_sem):
    c = lax.axis_index("c")
    s = lax.axis_index("s")
    wid = c * 16 + s

    @pl.when(s == 0)
    def _():
        pltpu.sync_copy(zeros_hbm, acc_sh)

    idx_cp = pltpu.async_copy(idx_hbm.at[wid], idx_v, ld_sem)
    vals_cp = pltpu.async_copy(vals_hbm.at[wid], vals_v, ld_sem)
    idx_cp.wait()
    vals_cp.wait()
    plsc.subcore_barrier()
    copies = [
        pltpu.async_copy(vals_v.at[pl.ds(j * IDX_CH, IDX_CH)],
                         acc_sh.at[idx_v.at[j]], sc_sem, add=True)
        for j in range(N_IDX_CH)
    ]
    for cp in copies:
        cp.wait()
    plsc.subcore_barrier()

    @pl.when(s == 0)
    def _():
        pltpu.sync_copy(acc_sh, out_hbm.at[c])


def _post_body(p_ref, m8_ref, amat_ref, bvec_ref, out_ref):
    mol = p_ref[0] + p_ref[1]
    dn = jnp.sqrt((mol * mol) @ m8_ref[...] + 1e-12)
    out_ref[...] = mol @ amat_ref[...] + dn @ bvec_ref[...]


_post_call = pl.pallas_call(
    _post_body,
    out_shape=jax.ShapeDtypeStruct((N_MOL, 9), jnp.float32),
)


def kernel(x_scalar, x_spherical, coord, batch_index, sw1, sb1, sw2, sb2,
           pw0, pb0, pw2, qw0, qb0, qw2):
    del coord
    eye5 = np.eye(5, dtype=np.float32)
    w2big = jnp.kron(pw2 * (1.0 / math.sqrt(32.0)), eye5)
    w2pad = jnp.zeros((256, 80), jnp.float32).at[64:224].set(w2big)
    q2big = jnp.kron(qw2 * (1.0 / math.sqrt(16.0)), eye5)

    atom = _dense_call(x_scalar, x_spherical, x_spherical,
                       sw1, sw2, pw0, w2pad, jnp.asarray(_SMAT),
                       jnp.asarray(_SMAT_T), qw0, q2big,
                       sb1.reshape(1, 64), sb2.reshape(1, 2),
                       pb0.reshape(1, 64), qb0.reshape(1, 1))

    idx_pad = jnp.zeros((NPAD,), jnp.int32).at[:N_ATOMS].set(batch_index)
    partials = _segsum(atom.reshape(NW, CHUNK, 8),
                       idx_pad.reshape(NW, N_IDX_CH, IDX_CH),
                       jnp.zeros((N_MOL, 8), jnp.float32))

    out9 = _post_call(partials, jnp.asarray(_M8), jnp.asarray(_AMAT),
                      jnp.asarray(_BVEC))
    return out9.reshape(N_MOL, 3, 3)


# speedup vs baseline: 1.0130x; 1.0130x over previous
import functools
import math

import jax
import jax.numpy as jnp
import numpy as np
from jax import lax
from jax.experimental import pallas as pl
from jax.experimental.pallas import tpu as pltpu
from jax.experimental.pallas import tpu_sc as plsc

N_ATOMS = 100000
N_MOL = 4096
SQ3 = 1.0 / math.sqrt(3.0)

NW = 32
NPAD = 102400
CHUNK = NPAD // NW
BN = 6400
NBLK = NPAD // BN
IDX_CH = 128
N_IDX_CH = CHUNK // IDX_CH

_SMAT = np.kron(np.eye(16, dtype=np.float32), np.ones((5, 1), np.float32))
_SMAT_T = np.ascontiguousarray(_SMAT.T)

_M8 = np.zeros((8, 1), np.float32)
_M8[1:6, 0] = 1.0
_AMAT = np.zeros((8, 9), np.float32)
_AMAT[0, [0, 4, 8]] = 1.0
_AMAT[1, [1, 3]] = 1.0
_AMAT[2, [5, 7]] = 1.0
_AMAT[3, [0, 4, 8]] = [-SQ3, -SQ3, 2.0 * SQ3]
_AMAT[4, [2, 6]] = 1.0
_AMAT[5, [0, 4]] = [1.0, -1.0]
_BVEC = np.zeros((1, 9), np.float32)
_BVEC[0, [0, 4, 8]] = SQ3


def _dense_body(xs_ref, x0_ref, x2_ref, sw1_ref, sw2_ref, pw0_ref,
                w2pad_ref, s_ref, st_ref, qw0_ref, q2big_ref,
                sb1_ref, sb2_ref, pb0_ref, qb0_ref, out_ref):
    pid = pl.program_id(0)

    h = xs_ref[...] @ sw1_ref[...] + sb1_ref[...]
    h = h * jax.nn.sigmoid(h)
    so = h @ sw2_ref[...] + sb2_ref[...]

    h0 = (x0_ref[...] @ pw0_ref[...]) * (1.0 / math.sqrt(128.0)) + pb0_ref[...]
    h0 = h0 * jax.nn.sigmoid(jnp.abs(h0))

    lane = lax.broadcasted_iota(jnp.int32, (BN, 256), 1)
    x2 = jnp.where(lane < 224, x2_ref[...], 0.0)
    h2 = x2 @ w2pad_ref[...]
    nsq = (h2 * h2) @ s_ref[...]
    g = jax.nn.sigmoid(jnp.sqrt(nsq + 1e-12))
    h2 = h2 * (g @ st_ref[...])

    o0 = (h0 @ qw0_ref[...]) * (1.0 / math.sqrt(64.0)) + qb0_ref[...]
    o2 = h2 @ q2big_ref[...]
    a0 = o0 * so[:, 0:1]
    a2 = o2 * so[:, 1:2]
    out = jnp.concatenate(
        [a0, a2, jnp.zeros((BN, 2), jnp.float32)], axis=-1)
    row = pid * BN + lax.broadcasted_iota(jnp.int32, (BN, 8), 0)
    out_ref[...] = jnp.where(row < N_ATOMS, out, 0.0)


def _whole(shape):
    return pl.BlockSpec(shape, lambda i: tuple(0 for _ in shape))


_dense_call = pl.pallas_call(
    _dense_body,
    grid=(NBLK,),
    in_specs=[
        pl.BlockSpec((BN, 128), lambda i: (i, 0)),
        pl.BlockSpec((BN, 128), lambda i: (i, 0)),
        pl.BlockSpec((BN, 256), lambda i: (i, 1)),
        _whole((128, 64)),
        _whole((64, 2)),
        _whole((128, 64)),
        _whole((256, 80)),
        _whole((80, 16)),
        _whole((16, 80)),
        _whole((64, 1)),
        _whole((80, 5)),
        _whole((1, 64)),
        _whole((1, 2)),
        _whole((1, 64)),
        _whole((1, 1)),
    ],
    out_specs=pl.BlockSpec((BN, 8), lambda i: (i, 0)),
    out_shape=jax.ShapeDtypeStruct((NPAD, 8), jnp.float32),
)


@functools.partial(
    pl.kernel,
    out_type=jax.ShapeDtypeStruct((2, N_MOL, 8), jnp.float32),
    mesh=plsc.VectorSubcoreMesh(core_axis_name="c", subcore_axis_name="s"),
    compiler_params=pltpu.CompilerParams(use_tc_tiling_on_sc=False),
    scratch_types=[
        pltpu.VMEM((N_IDX_CH, IDX_CH), jnp.int32),
        pltpu.VMEM((CHUNK, 8), jnp.float32),
        pltpu.VMEM_SHARED((N_MOL, 8), jnp.float32),
        pltpu.SemaphoreType.DMA,
        pltpu.SemaphoreType.DMA,
    ],
)
def _segsum(vals_hbm, idx_hbm, zeros_hbm, out_hbm, idx_v, vals_v, acc_sh,
            ld_sem, sc_sem):
    c = lax.axis_index("c")
    s = lax.axis_index("s")
    wid = c * 16 + s

    @pl.when(s == 0)
    def _():
        pltpu.sync_copy(zeros_hbm, acc_sh)

    idx_cp = pltpu.async_copy(idx_hbm.at[wid], idx_v, ld_sem)
    vals_cp = pltpu.async_copy(vals_hbm.at[wid], vals_v, ld_sem)
    idx_cp.wait()
    vals_cp.wait()
    plsc.subcore_barrier()
    copies = [
        pltpu.async_copy(vals_v.at[pl.ds(j * IDX_CH, IDX_CH)],
                         acc_sh.at[idx_v.at[j]], sc_sem, add=True)
        for j in range(N_IDX_CH)
    ]
    for cp in copies:
        cp.wait()
    plsc.subcore_barrier()

    @pl.when(s == 0)
    def _():
        pltpu.sync_copy(acc_sh, out_hbm.at[c])


def _post_body(p_ref, m8_ref, amat_ref, bvec_ref, out_ref):
    mol = p_ref[0] + p_ref[1]
    dn = jnp.sqrt((mol * mol) @ m8_ref[...] + 1e-12)
    out_ref[...] = mol @ amat_ref[...] + dn @ bvec_ref[...]


_post_call = pl.pallas_call(
    _post_body,
    out_shape=jax.ShapeDtypeStruct((N_MOL, 9), jnp.float32),
)


def kernel(x_scalar, x_spherical, coord, batch_index, sw1, sb1, sw2, sb2,
           pw0, pb0, pw2, qw0, qb0, qw2):
    del coord
    eye5 = np.eye(5, dtype=np.float32)
    w2big = jnp.kron(pw2 * (1.0 / math.sqrt(32.0)), eye5)
    w2pad = jnp.zeros((256, 80), jnp.float32).at[64:224].set(w2big)
    q2big = jnp.kron(qw2 * (1.0 / math.sqrt(16.0)), eye5)

    atom = _dense_call(x_scalar, x_spherical, x_spherical,
                       sw1, sw2, pw0, w2pad, jnp.asarray(_SMAT),
                       jnp.asarray(_SMAT_T), qw0, q2big,
                       sb1.reshape(1, 64), sb2.reshape(1, 2),
                       pb0.reshape(1, 64), qb0.reshape(1, 1))

    idx_pad = jnp.zeros((NPAD,), jnp.int32).at[:N_ATOMS].set(batch_index)
    partials = _segsum(atom.reshape(NW, CHUNK, 8),
                       idx_pad.reshape(NW, N_IDX_CH, IDX_CH),
                       jnp.zeros((N_MOL, 8), jnp.float32))

    out9 = _post_call(partials, jnp.asarray(_M8), jnp.asarray(_AMAT),
                      jnp.asarray(_BVEC))
    return out9.reshape(N_MOL, 3, 3)
